# baseline (speedup 1.0000x reference)
import functools

import jax
import jax.numpy as jnp
from jax import lax
from jax.experimental import pallas as pl
from jax.experimental.pallas import tpu as pltpu

F32 = jnp.float32
OFFS = tuple((di, dj) for di in (-1, 0, 1) for dj in (-1, 0, 1))

_INTERPRET = False


def _shift9(x, w9, b, W):
    HW, Cin = x.shape
    Cout = w9.shape[-1]
    col = lax.broadcasted_iota(jnp.int32, (HW, 1), 0) % W
    acc = jnp.broadcast_to(b, (HW, Cout)).astype(F32)
    for t, (di, dj) in enumerate(OFFS):
        s = di * W + dj
        lo = max(0, -s)
        hi = HW - max(0, s)
        xs = x[lo + s:hi + s, :]
        if dj < 0:
            xs = jnp.where(col[lo:hi] >= -dj, xs, 0.0)
        elif dj > 0:
            xs = jnp.where(col[lo:hi] < W - dj, xs, 0.0)
        c = jnp.dot(xs, w9[t], preferred_element_type=F32)
        if lo > 0:
            c = jnp.concatenate([jnp.zeros((lo, Cout), F32), c], axis=0)
        if hi < HW:
            c = jnp.concatenate([c, jnp.zeros((HW - hi, Cout), F32)], axis=0)
        acc = acc + c
    return acc


def _mm(x, w, b):
    return jnp.dot(x, w, preferred_element_type=F32) + b


def _relu(x):
    return jnp.maximum(x, 0.0)


def _conv9_body(W, x_ref, w9_ref, b_ref, out_ref):
    out_ref[0] = _relu(_shift9(x_ref[0], w9_ref[...], b_ref[...], W))


def _resblock(x, wr, W, shortcut):
    c1w, c1b, c2w9, c2b, c3w, c3b = (r[...] for r in wr)
    h = _relu(_mm(x, c1w, c1b))
    h = _relu(_shift9(h, c2w9, c2b, W))
    h = _relu(_mm(h, c3w, c3b))
    return x + h if shortcut else h


def _enc_body(x_ref, *refs):
    out_ref = refs[-1]
    wr = refs[:-1]
    x = x_ref[0]
    for i in range(4):
        x = _resblock(x, wr[i * 6:(i + 1) * 6], 56, True)
    out_ref[0] = _mm(x, wr[24][...], wr[25][...])


def _dec1_body(x_ref, *refs):
    out_ref = refs[-1]
    wr = refs[:-1]
    x = _resblock(x_ref[0], wr[0:6], 56, False)
    x = _resblock(x, wr[6:12], 56, True)
    out_ref[0] = _relu(_shift9(x, wr[12][...], wr[13][...], 56))


def _rb_body(W, shortcut, x_ref, *refs):
    out_ref = refs[-1]
    out_ref[0] = _resblock(x_ref[0], refs[:-1], W, shortcut)


def _ext_rows(src, r0, r1, PAD, HW, C):
    parts = []
    if r0 == 0:
        parts.append(jnp.zeros((PAD, C), F32))
    else:
        parts.append(src[r0 - PAD:r0])
    parts.append(src[r0:r1])
    if r1 == HW:
        parts.append(jnp.zeros((PAD, C), F32))
    else:
        parts.append(src[r1:r1 + PAD])
    return jnp.concatenate(parts, axis=0)


def _shift9_chunk(xe, w9, b, W, PAD, chunk):
    Cout = w9.shape[-1]
    col = lax.broadcasted_iota(jnp.int32, (chunk, 1), 0) % W
    acc = jnp.broadcast_to(b, (chunk, Cout)).astype(F32)
    for t, (di, dj) in enumerate(OFFS):
        s = di * W + dj
        xs = xe[PAD + s:PAD + chunk + s, :]
        if dj < 0:
            xs = jnp.where(col >= -dj, xs, 0.0)
        elif dj > 0:
            xs = jnp.where(col < W - dj, xs, 0.0)
        acc = acc + jnp.dot(xs, w9[t], preferred_element_type=F32)
    return acc


def _rb_body_big(W, nchunks, x_ref, c1w_r, c1b_r, c2w9_r, c2b_r, c3w_r,
                 c3b_r, out_ref, h1_ref):
    HW = x_ref.shape[1]
    chunk = HW // nchunks
    PAD = W + 8
    c1w, c1b = c1w_r[...], c1b_r[...]
    for c in range(nchunks):
        r0 = c * chunk
        h1_ref[r0:r0 + chunk] = _relu(_mm(x_ref[0, r0:r0 + chunk], c1w, c1b))
    w9, c2b, c3w, c3b = c2w9_r[...], c2b_r[...], c3w_r[...], c3b_r[...]
    Cm = w9.shape[1]
    for c in range(nchunks):
        r0, r1 = c * chunk, (c + 1) * chunk
        xe = _ext_rows(h1_ref, r0, r1, PAD, HW, Cm)
        h2 = _relu(_shift9_chunk(xe, w9, c2b, W, PAD, chunk))
        out_ref[0, r0:r1] = x_ref[0, r0:r1] + _relu(_mm(h2, c3w, c3b))


def _conv9_body_big(W, nchunks, x_ref, w9_ref, b_ref, out_ref):
    HW = x_ref.shape[1]
    chunk = HW // nchunks
    PAD = W + 8
    w9, b = w9_ref[...], b_ref[...]
    Cin = w9.shape[1]
    for c in range(nchunks):
        r0, r1 = c * chunk, (c + 1) * chunk
        xe = _ext_rows(x_ref.at[0], r0, r1, PAD, HW, Cin)
        out_ref[0, r0:r1] = _relu(_shift9_chunk(xe, w9, b, W, PAD, chunk))


def _vq_body(z_ref, emb_ref, esq_ref, zq_ref):
    z = z_ref[...]
    emb = emb_ref[...]
    mm = lax.dot_general(z, emb, (((1,), (1,)), ((), ())),
                         preferred_element_type=F32)
    zsq = jnp.sum(z * z, axis=1, keepdims=True)
    dist = zsq + esq_ref[...] - 2.0 * mm
    m = jnp.min(dist, axis=1, keepdims=True)
    iota2 = lax.broadcasted_iota(jnp.int32, dist.shape, 1)
    cand = jnp.where(dist == m, iota2, dist.shape[1])
    idx = jnp.min(cand, axis=1, keepdims=True)
    oh = (iota2 == idx).astype(F32)
    zq_ref[...] = jnp.dot(oh, emb, preferred_element_type=F32)


def _pack_3x3(w):
    return jnp.stack([w[:, :, di + 1, dj + 1].T for (di, dj) in OFFS])


def _pack_s2(w):
    Co, Ci = w.shape[0], w.shape[1]
    taps = []
    for di in (-1, 0, 1):
        for dj in (-1, 0, 1):
            blk = jnp.zeros((2, 2, Ci, Co), F32)
            for pi in (0, 1):
                ky = 2 * di + pi + 1
                if 0 <= ky < 4:
                    for pj in (0, 1):
                        kx = 2 * dj + pj + 1
                        if 0 <= kx < 4:
                            blk = blk.at[pi, pj].set(w[:, :, ky, kx].T)
            taps.append(blk.reshape(4 * Ci, Co))
    return jnp.stack(taps)


def _pack_t2(w):
    Ci, Co = w.shape[0], w.shape[1]
    taps = []
    for di in (-1, 0, 1):
        for dj in (-1, 0, 1):
            blk = jnp.zeros((Ci, 2, 2, Co), F32)
            for r in (0, 1):
                ki = -2 * di + r + 1
                if 0 <= ki < 4:
                    for s in (0, 1):
                        kj = -2 * dj + s + 1
                        if 0 <= kj < 4:
                            blk = blk.at[:, r, s, :].set(w[:, :, ki, kj])
            taps.append(blk.reshape(Ci, 4 * Co))
    return jnp.stack(taps)


def _rb_weights(p, pre):
    return [p[pre + 'c1_w'][:, :, 0, 0].T, p[pre + 'c1_b'][None],
            _pack_3x3(p[pre + 'c2_w']), p[pre + 'c2_b'][None],
            p[pre + 'c3_w'][:, :, 0, 0].T, p[pre + 'c3_b'][None]]


def _img_call(body, x, ws, HW, Cout, scratch=()):
    N = x.shape[0]
    in_specs = [pl.BlockSpec((1,) + x.shape[1:], lambda n: (n, 0, 0))]
    for w in ws:
        in_specs.append(
            pl.BlockSpec(w.shape, functools.partial(lambda nd, n: (0,) * nd, w.ndim)))
    return pl.pallas_call(
        body, grid=(N,), in_specs=in_specs,
        out_specs=pl.BlockSpec((1, HW, Cout), lambda n: (n, 0, 0)),
        out_shape=jax.ShapeDtypeStruct((N, HW, Cout), F32),
        scratch_shapes=list(scratch),
        interpret=_INTERPRET)(x, *ws)


def _vq_call(zf, emb, esq):
    M = zf.shape[0]
    BLK = 512 if M % 512 == 0 else 392
    return pl.pallas_call(
        _vq_body, grid=(M // BLK,),
        in_specs=[pl.BlockSpec((BLK, 64), lambda i: (i, 0)),
                  pl.BlockSpec(emb.shape, lambda i: (0, 0)),
                  pl.BlockSpec(esq.shape, lambda i: (0, 0))],
        out_specs=pl.BlockSpec((BLK, 64), lambda i: (i, 0)),
        out_shape=jax.ShapeDtypeStruct((M, 64), F32),
        interpret=_INTERPRET)(zf, emb, esq)


def kernel(x, params):
    p = params
    N = x.shape[0]

    xp = (x.transpose(0, 2, 3, 1).reshape(N, 112, 2, 112, 2, 3)
          .transpose(0, 1, 3, 2, 4, 5).reshape(N, 112 * 112, 12))
    h = _img_call(functools.partial(_conv9_body_big, 112, 4), xp,
                  [_pack_s2(p['e1_w']), p['e1_b'][None]], 112 * 112, 64)

    hp = (h.reshape(N, 56, 2, 56, 2, 64)
          .transpose(0, 1, 3, 2, 4, 5).reshape(N, 56 * 56, 256))
    h = _img_call(functools.partial(_conv9_body, 56), hp,
                  [_pack_s2(p['e2_w']), p['e2_b'][None]], 56 * 56, 128)

    enc_ws = []
    for s in ('s3', 's4'):
        for bl in ('b0', 'b1'):
            enc_ws += _rb_weights(p, 'e_' + s + '_' + bl + '_')
    enc_ws += [p['proj_w'][:, :, 0, 0].T, p['proj_b'][None]]
    z = _img_call(_enc_body, h, enc_ws, 56 * 56, 64)

    emb = p['emb']
    esq = jnp.sum(emb ** 2, axis=1)[None]
    zq = _vq_call(z.reshape(N * 56 * 56, 64), emb, esq).reshape(N, 56 * 56, 64)

    dec1_ws = _rb_weights(p, 'd_s1_b0_') + _rb_weights(p, 'd_s1_b1_')
    dec1_ws += [_pack_t2(p['d2_w']), jnp.tile(p['d2_b'], 4)[None]]
    y = _img_call(_dec1_body, zq, dec1_ws, 56 * 56, 256)
    y = (y.reshape(N, 56, 56, 2, 2, 64)
         .transpose(0, 1, 3, 2, 4, 5).reshape(N, 112 * 112, 64))

    rb_scr = (pltpu.VMEM((112 * 112, 32), F32),)
    y = _img_call(functools.partial(_rb_body_big, 112, 4), y,
                  _rb_weights(p, 'd_s3_b0_'), 112 * 112, 64, rb_scr)
    y = _img_call(functools.partial(_rb_body_big, 112, 4), y,
                  _rb_weights(p, 'd_s3_b1_'), 112 * 112, 64, rb_scr)
    out = _img_call(functools.partial(_conv9_body_big, 112, 4), y,
                    [_pack_t2(p['d4_w']), jnp.tile(p['d4_b'], 4)[None]],
                    112 * 112, 12)

    out = (out.reshape(N, 112, 112, 2, 2, 3)
           .transpose(0, 1, 3, 2, 4, 5).reshape(N, 224, 224, 3)
           .transpose(0, 3, 1, 2))
    return out


# speedup vs baseline: 1.7173x; 1.7173x over previous
import functools

import jax
import jax.numpy as jnp
from jax import lax
from jax.experimental import pallas as pl
from jax.experimental.pallas import tpu as pltpu

F32 = jnp.float32
OFFS = tuple((di, dj) for di in (-1, 0, 1) for dj in (-1, 0, 1))

_INTERPRET = False


def _shift9(x, w9, b, W):
    HW, Cin = x.shape
    Cout = w9.shape[-1]
    col = lax.broadcasted_iota(jnp.int32, (HW, 1), 0) % W
    acc = jnp.broadcast_to(b, (HW, Cout)).astype(F32)
    for t, (di, dj) in enumerate(OFFS):
        s = di * W + dj
        lo = max(0, -s)
        hi = HW - max(0, s)
        xs = x[lo + s:hi + s, :]
        if dj < 0:
            xs = jnp.where(col[lo:hi] >= -dj, xs, 0.0)
        elif dj > 0:
            xs = jnp.where(col[lo:hi] < W - dj, xs, 0.0)
        c = jnp.dot(xs, w9[t], preferred_element_type=F32)
        if lo > 0:
            c = jnp.concatenate([jnp.zeros((lo, Cout), F32), c], axis=0)
        if hi < HW:
            c = jnp.concatenate([c, jnp.zeros((HW - hi, Cout), F32)], axis=0)
        acc = acc + c
    return acc


def _mm(x, w, b):
    return jnp.dot(x, w, preferred_element_type=F32) + b


def _relu(x):
    return jnp.maximum(x, 0.0)


def _resblock(x, wr, W, shortcut):
    c1w, c1b, c2w9, c2b, c3w, c3b = (r[...] for r in wr)
    h = _relu(_mm(x, c1w, c1b))
    h = _relu(_shift9(h, c2w9, c2b, W))
    h = _relu(_mm(h, c3w, c3b))
    return x + h if shortcut else h


def _enc_body(x_ref, *refs):
    out_ref = refs[-1]
    wr = refs[:-1]
    h = _relu(_shift9(x_ref[0], wr[0][...], wr[1][...], 56))
    h = _relu(_shift9(h, wr[2][...], wr[3][...], 56))
    for i in range(4):
        h = _resblock(h, wr[4 + i * 6:10 + i * 6], 56, True)
    out_ref[0] = _mm(h, wr[28][...], wr[29][...])


def _dec_body(x_ref, *refs):
    out_ref = refs[-1]
    wr = refs[:-1]
    x = _resblock(x_ref[0], wr[0:6], 56, False)
    x = _resblock(x, wr[6:12], 56, True)
    x = _relu(_shift9(x, wr[12][...], wr[13][...], 56))
    x = _resblock(x, wr[14:20], 56, True)
    x = _resblock(x, wr[20:26], 56, True)
    out_ref[0] = _relu(_shift9(x, wr[26][...], wr[27][...], 56))


def _vq_body(z_ref, emb_ref, esq_ref, zq_ref):
    z = z_ref[...]
    emb = emb_ref[...]
    mm = lax.dot_general(z, emb, (((1,), (1,)), ((), ())),
                         preferred_element_type=F32)
    zsq = jnp.sum(z * z, axis=1, keepdims=True)
    dist = zsq + esq_ref[...] - 2.0 * mm
    m = jnp.min(dist, axis=1, keepdims=True)
    iota2 = lax.broadcasted_iota(jnp.int32, dist.shape, 1)
    cand = jnp.where(dist == m, iota2, dist.shape[1])
    idx = jnp.min(cand, axis=1, keepdims=True)
    oh = (iota2 == idx).astype(F32)
    zq_ref[...] = jnp.dot(oh, emb, preferred_element_type=F32)


def _pack_pp(w, Pin, Pout, K, rel, transposed=False):
    if transposed:
        Ci, Co = w.shape[0], w.shape[1]
        get = lambda ki, kj: w[:, :, ki, kj]
    else:
        Co, Ci = w.shape[0], w.shape[1]
        get = lambda ki, kj: w[:, :, ki, kj].T
    taps = []
    for bdi in (-1, 0, 1):
        for bdj in (-1, 0, 1):
            blk = jnp.zeros((Pin, Pin, Ci, Pout, Pout, Co), F32)
            for qi in range(Pin):
                for ri in range(Pout):
                    ki = rel(bdi, qi, ri)
                    if not 0 <= ki < K:
                        continue
                    for qj in range(Pin):
                        for rj in range(Pout):
                            kj = rel(bdj, qj, rj)
                            if 0 <= kj < K:
                                blk = blk.at[qi, qj, :, ri, rj, :].set(get(ki, kj))
            taps.append(blk.reshape(Pin * Pin * Ci, Pout * Pout * Co))
    return jnp.stack(taps)


def _pack_3x3(w):
    return jnp.stack([w[:, :, di + 1, dj + 1].T for (di, dj) in OFFS])


def _pack_1x1_pp(wmat, P2):
    Ci, Co = wmat.shape
    W = jnp.zeros((P2, Ci, P2, Co), F32)
    for t in range(P2):
        W = W.at[t, :, t, :].set(wmat)
    return W.reshape(P2 * Ci, P2 * Co)


def _rb_weights(p, pre):
    return [p[pre + 'c1_w'][:, :, 0, 0].T, p[pre + 'c1_b'][None],
            _pack_3x3(p[pre + 'c2_w']), p[pre + 'c2_b'][None],
            p[pre + 'c3_w'][:, :, 0, 0].T, p[pre + 'c3_b'][None]]


def _rb_weights_pp(p, pre):
    return [_pack_1x1_pp(p[pre + 'c1_w'][:, :, 0, 0].T, 4),
            jnp.tile(p[pre + 'c1_b'], 4)[None],
            _pack_pp(p[pre + 'c2_w'], 2, 2, 3,
                     lambda bd, q, r: 2 * bd + q - r + 1),
            jnp.tile(p[pre + 'c2_b'], 4)[None],
            _pack_1x1_pp(p[pre + 'c3_w'][:, :, 0, 0].T, 4),
            jnp.tile(p[pre + 'c3_b'], 4)[None]]


def _img_call(body, x, ws, HW, Cout, scratch=()):
    N = x.shape[0]
    in_specs = [pl.BlockSpec((1,) + x.shape[1:], lambda n: (n, 0, 0))]
    for w in ws:
        in_specs.append(
            pl.BlockSpec(w.shape, functools.partial(lambda nd, n: (0,) * nd, w.ndim)))
    return pl.pallas_call(
        body, grid=(N,), in_specs=in_specs,
        out_specs=pl.BlockSpec((1, HW, Cout), lambda n: (n, 0, 0)),
        out_shape=jax.ShapeDtypeStruct((N, HW, Cout), F32),
        scratch_shapes=list(scratch),
        interpret=_INTERPRET)(x, *ws)


def _vq_call(zf, emb, esq):
    M = zf.shape[0]
    BLK = 512 if M % 512 == 0 else 392
    return pl.pallas_call(
        _vq_body, grid=(M // BLK,),
        in_specs=[pl.BlockSpec((BLK, 64), lambda i: (i, 0)),
                  pl.BlockSpec(emb.shape, lambda i: (0, 0)),
                  pl.BlockSpec(esq.shape, lambda i: (0, 0))],
        out_specs=pl.BlockSpec((BLK, 64), lambda i: (i, 0)),
        out_shape=jax.ShapeDtypeStruct((M, 64), F32),
        interpret=_INTERPRET)(zf, emb, esq)


def kernel(x, params):
    p = params
    N = x.shape[0]

    xp = (x.transpose(0, 2, 3, 1).reshape(N, 56, 4, 56, 4, 3)
          .transpose(0, 1, 3, 2, 4, 5).reshape(N, 56 * 56, 48))

    enc_ws = [_pack_pp(p['e1_w'], 4, 2, 4,
                       lambda bd, q, r: 4 * bd + q - 2 * r + 1),
              jnp.tile(p['e1_b'], 4)[None],
              _pack_pp(p['e2_w'], 2, 1, 4,
                       lambda bd, q, r: 2 * bd + q + 1),
              p['e2_b'][None]]
    for s in ('s3', 's4'):
        for bl in ('b0', 'b1'):
            enc_ws += _rb_weights(p, 'e_' + s + '_' + bl + '_')
    enc_ws += [p['proj_w'][:, :, 0, 0].T, p['proj_b'][None]]
    z = _img_call(_enc_body, xp, enc_ws, 56 * 56, 64)

    emb = p['emb']
    esq = jnp.sum(emb ** 2, axis=1)[None]
    zq = _vq_call(z.reshape(N * 56 * 56, 64), emb, esq).reshape(N, 56 * 56, 64)

    dec_ws = _rb_weights(p, 'd_s1_b0_') + _rb_weights(p, 'd_s1_b1_')
    dec_ws += [_pack_pp(p['d2_w'], 1, 2, 4,
                        lambda bd, q, r: -2 * bd + r + 1, transposed=True),
               jnp.tile(p['d2_b'], 4)[None]]
    dec_ws += _rb_weights_pp(p, 'd_s3_b0_') + _rb_weights_pp(p, 'd_s3_b1_')
    dec_ws += [_pack_pp(p['d4_w'], 2, 4, 4,
                        lambda bd, q, r: r - 2 * q - 4 * bd + 1, transposed=True),
               jnp.tile(p['d4_b'], 16)[None]]
    out = _img_call(_dec_body, zq, dec_ws, 56 * 56, 48)

    out = (out.reshape(N, 56, 56, 4, 4, 3)
           .transpose(0, 1, 3, 2, 4, 5).reshape(N, 224, 224, 3)
           .transpose(0, 3, 1, 2))
    return out
